# baseline (speedup 1.0000x reference)
import functools

import jax
import jax.numpy as jnp
from jax import lax
from jax.experimental import pallas as pl
from jax.experimental.pallas import tpu as pltpu
from jax.experimental.pallas import tpu_sc as plsc

N = 18432
D = 64
K = 1024
R = 1024
NB = N // R

_NC = 2
_NS = 16
NW = _NC * _NS
BPW = N // NW


def _tc_body(x_ref, emb_ref, enc_ref, idx_ref):
    x = x_ref[...]
    emb = emb_ref[...]
    sim = jnp.dot(x, emb, preferred_element_type=jnp.float32)
    e2 = jnp.sum(emb * emb, axis=0, keepdims=True)
    scores = e2 - 2.0 * sim
    idx = jnp.argmin(scores, axis=1).astype(jnp.int32)
    cols = lax.broadcasted_iota(jnp.int32, (R, K), 1)
    enc_ref[...] = jnp.where(cols == idx[:, None], 1.0, 0.0)
    idx_ref[...] = idx.reshape(1, 1, R)


_tc_call = pl.pallas_call(
    _tc_body,
    grid=(NB,),
    in_specs=[
        pl.BlockSpec((R, D), lambda i: (i, 0)),
        pl.BlockSpec((D, K), lambda i: (0, 0)),
    ],
    out_specs=[
        pl.BlockSpec((R, K), lambda i: (i, 0)),
        pl.BlockSpec((1, 1, R), lambda i: (i, 0, 0)),
    ],
    out_shape=[
        jax.ShapeDtypeStruct((N, K), jnp.float32),
        jax.ShapeDtypeStruct((NB, 1, R), jnp.int32),
    ],
    compiler_params=pltpu.CompilerParams(
        dimension_semantics=("arbitrary",),
    ),
)

@functools.cache
def _make_sc_gather():
    mesh = plsc.VectorSubcoreMesh(
        core_axis_name="c", subcore_axis_name="s", num_cores=_NC)

    @functools.partial(
        pl.kernel,
        mesh=mesh,
        out_type=jax.ShapeDtypeStruct((N, D), jnp.float32),
        scratch_types=[
            pltpu.VMEM((BPW,), jnp.int32),
            pltpu.VMEM((BPW, D), jnp.float32),
            pltpu.SemaphoreType.DMA,
        ],
        compiler_params=pltpu.CompilerParams(use_tc_tiling_on_sc=False),
    )
    def _sc_gather(table_hbm, idx_hbm, out_hbm, idx_v, rows_v, sem):
        wid = lax.axis_index("s") * _NC + lax.axis_index("c")
        base = wid * BPW
        pltpu.sync_copy(idx_hbm.at[pl.ds(base, BPW)], idx_v)
        pltpu.async_copy(table_hbm.at[idx_v], rows_v, sem).wait()
        pltpu.sync_copy(rows_v, out_hbm.at[pl.ds(base, BPW)])

    return _sc_gather


def kernel(x, embeddings):
    encodings, idx3 = _tc_call(x, embeddings)
    table = embeddings.T
    quantized = _make_sc_gather()(table, idx3.reshape(N))
    return (encodings, quantized)
